# Initial kernel scaffold; baseline (speedup 1.0000x reference)
import jax
import jax.numpy as jnp
from jax.experimental import pallas as pl


def kernel(input):
    raise NotImplementedError("write your pallas kernel here")


# speedup vs baseline: 1.7436x; 1.7436x over previous
import functools

import jax
import jax.numpy as jnp
from jax import lax
from jax.experimental import pallas as pl
from jax.experimental.pallas import tpu as pltpu
from jax.experimental.pallas import tpu_sc as plsc

_NBFEAT = 128
_L = 16
_NC = 2
_NS = 16
_NW = _NC * _NS
_PREFIX = 896
_R = 32
_NGROUPS = _NBFEAT // _L


def _sc_qtransform(x2, il, ih, w, n_rows):
    rows_per_w = n_rows // _NW
    n_chunks = rows_per_w // _R
    mesh = plsc.VectorSubcoreMesh(
        core_axis_name="c", subcore_axis_name="s",
        num_cores=_NC, num_subcores=_NS)

    def body(x_hbm, il_hbm, ih_hbm, w_hbm, out_hbm,
             il_v, ih_v, w_v, buf0, buf1, ob0, ob1,
             isem0, isem1, osem0, osem1):
        wid = lax.axis_index("c") * _NS + lax.axis_index("s")
        base = wid * rows_per_w

        pltpu.sync_copy(il_hbm, il_v)
        pltpu.sync_copy(ih_hbm, ih_v)
        pltpu.sync_copy(w_hbm, w_v)

        ilv = [il_v[pl.ds(g * _L, _L)] for g in range(_NGROUPS)]
        ihv = [ih_v[pl.ds(g * _L, _L)] for g in range(_NGROUPS)]
        wv = [w_v[pl.ds(g * _L, _L)] for g in range(_NGROUPS)]
        cwv = [1.0 - wv[g] for g in range(_NGROUPS)]

        bufs = [buf0, buf1]
        obs = [ob0, ob1]
        isems = [isem0, isem1]
        osems = [osem0, osem1]

        def in_copy(c):
            return pltpu.make_async_copy(
                x_hbm.at[pl.ds(base + c * _R, _R), pl.ds(0, _PREFIX)],
                bufs[c % 2], isems[c % 2])

        def out_copy(c):
            return pltpu.make_async_copy(
                obs[c % 2], out_hbm.at[pl.ds(base + c * _R, _R)],
                osems[c % 2])

        in_copy(0).start()
        for c in range(n_chunks):
            cur = c % 2
            if c + 1 < n_chunks:
                in_copy(c + 1).start()
            in_copy(c).wait()
            if c >= 2:
                out_copy(c - 2).wait()
            buf, ob = bufs[cur], obs[cur]

            def row_body(r, _):
                rsplat = jnp.full((_L,), r, dtype=jnp.int32)
                for g in range(_NGROUPS):
                    lo = plsc.load_gather(buf, [rsplat, ilv[g]])
                    hi = plsc.load_gather(buf, [rsplat, ihv[g]])
                    ob[r, pl.ds(g * _L, _L)] = hi * wv[g] + lo * cwv[g]
                return 0

            lax.fori_loop(0, _R, row_body, 0)
            out_copy(c).start()
        out_copy(n_chunks - 2).wait()
        out_copy(n_chunks - 1).wait()

    call = pl.kernel(
        body,
        out_type=jax.ShapeDtypeStruct((n_rows, _NBFEAT), jnp.float32),
        mesh=mesh,
        compiler_params=pltpu.CompilerParams(needs_layout_passes=False),
        scratch_types=[
            pltpu.VMEM((_NBFEAT,), jnp.int32),
            pltpu.VMEM((_NBFEAT,), jnp.int32),
            pltpu.VMEM((_NBFEAT,), jnp.float32),
            pltpu.VMEM((_R, _PREFIX), jnp.float32),
            pltpu.VMEM((_R, _PREFIX), jnp.float32),
            pltpu.VMEM((_R, _NBFEAT), jnp.float32),
            pltpu.VMEM((_R, _NBFEAT), jnp.float32),
            pltpu.SemaphoreType.DMA,
            pltpu.SemaphoreType.DMA,
            pltpu.SemaphoreType.DMA,
            pltpu.SemaphoreType.DMA,
        ],
    )
    return call(x2, il, ih, w)


def kernel(input):
    x = input
    b, t, c = x.shape
    n_rows = b * t
    halftone = jnp.float32(2.0 ** (1.0 / 12.0))
    f0 = jnp.float32(440.0 / 16000.0 * 1024.0)
    freq = f0 * jnp.power(halftone, jnp.arange(_NBFEAT, dtype=jnp.float32) - 69.0)
    lowfreq = jnp.floor(freq)
    w = freq - lowfreq
    il = lowfreq.astype(jnp.int32)
    ih = jnp.ceil(freq).astype(jnp.int32)
    x2 = x.reshape(n_rows, c)
    out2 = _sc_qtransform(x2, il, ih, w, n_rows)
    return out2.reshape(b, t, _NBFEAT)
